# baseline (speedup 1.0000x reference)
import functools

import jax
import jax.numpy as jnp
from jax import lax
from jax.experimental import pallas as pl
from jax.experimental.pallas import tpu as pltpu
from jax.experimental.pallas import tpu_sc as plsc

_XDIM = 64
_CAP = 1000000
_BSZ = 16384
_NC = 2
_NS = 16
_NW = _NC * _NS
_BPW = _BSZ // _NW
_LANES = 16
_NVEC = _BPW // _LANES
_CHUNK = 128
_NCHUNK = _BPW // _CHUNK
_EPW = _BPW * _XDIM
_NSTREAM = _EPW // _CHUNK

_mesh = plsc.VectorSubcoreMesh(core_axis_name="c", subcore_axis_name="s")


@functools.partial(
    pl.kernel,
    mesh=_mesh,
    out_type=(
        jax.ShapeDtypeStruct((_BSZ * _XDIM,), jnp.float32),
        jax.ShapeDtypeStruct((_BSZ,), jnp.int32),
    ),
    scratch_types=[
        pltpu.VMEM((_BPW,), jnp.int32),
        pltpu.VMEM((_EPW,), jnp.int32),
        pltpu.VMEM((_EPW,), jnp.float32),
        pltpu.VMEM((_BPW,), jnp.int32),
        pltpu.SemaphoreType.DMA,
        pltpu.SemaphoreType.DMA,
    ],
)
def _fetch(idx_hbm, mxf_hbm, my_hbm, out_x, out_y, idx_v, alist, rows_v,
           y_v, sem_x, sem_y):
    wid = lax.axis_index("s") * _NC + lax.axis_index("c")
    base = wid * _BPW
    pltpu.sync_copy(idx_hbm.at[pl.ds(base, _BPW)], idx_v)

    y_copies = []
    for j in range(_NCHUNK):
        y_copies.append(
            pltpu.async_copy(
                my_hbm.at[idx_v.at[pl.ds(j * _CHUNK, _CHUNK)]],
                y_v.at[pl.ds(j * _CHUNK, _CHUNK)],
                sem_y,
            )
        )

    def abody(k, carry):
        v = idx_v[pl.ds(k * _LANES, _LANES)]

        def fbody(f, c2):
            alist[pl.ds(f * _BPW + k * _LANES, _LANES)] = v + f * _CAP
            return c2

        lax.fori_loop(0, _XDIM, fbody, None)
        return carry

    lax.fori_loop(0, _NVEC, abody, None)

    def sbody(s, carry):
        pltpu.async_copy(
            mxf_hbm.at[alist.at[pl.ds(s * _CHUNK, _CHUNK)]],
            rows_v.at[pl.ds(s * _CHUNK, _CHUNK)],
            sem_x,
        )
        return carry

    lax.fori_loop(0, _NSTREAM, sbody, None)

    out_slice = out_x.at[pl.ds(wid * _EPW, _EPW)]
    pltpu.make_async_copy(out_slice, rows_v, sem_x).wait()
    for c in y_copies:
        c.wait()
    pltpu.sync_copy(rows_v, out_slice)
    pltpu.sync_copy(y_v, out_y.at[pl.ds(base, _BPW)])


def kernel(inputs, idx, mems_x, mems_y):
    del inputs
    mxf = mems_x.T.reshape(_XDIM * _CAP)
    out_xf, res_y = _fetch(idx, mxf, mems_y)
    res_x = (
        out_xf.reshape(_NW, _XDIM, _BPW)
        .transpose(0, 2, 1)
        .reshape(_BSZ, _XDIM)
    )
    return (res_x, res_y)


# speedup vs baseline: 8.3572x; 8.3572x over previous
import functools

import jax
import jax.numpy as jnp
from jax import lax
from jax.experimental import pallas as pl
from jax.experimental.pallas import tpu as pltpu
from jax.experimental.pallas import tpu_sc as plsc

_XDIM = 64
_PAD = 128
_CAP = 1000000
_BSZ = 16384
_NC = 2
_NS = 16
_NW = _NC * _NS
_BPW = _BSZ // _NW
_LANES = 16
_NVEC = _BPW // _LANES
_CHUNK = 128
_NCHUNK = _BPW // _CHUNK

_mesh = plsc.VectorSubcoreMesh(core_axis_name="c", subcore_axis_name="s")


@functools.partial(
    pl.kernel,
    mesh=_mesh,
    compiler_params=pltpu.CompilerParams(
        use_tc_tiling_on_sc=True, needs_layout_passes=False
    ),
    out_type=(
        jax.ShapeDtypeStruct((_BSZ * _XDIM,), jnp.float32),
        jax.ShapeDtypeStruct((_BSZ,), jnp.int32),
    ),
    scratch_types=[
        pltpu.VMEM((_BPW,), jnp.int32),
        pltpu.VMEM((_BPW, _PAD), jnp.float32),
        pltpu.VMEM((_BPW * _XDIM,), jnp.float32),
        pltpu.VMEM((_BPW,), jnp.int32),
        pltpu.SemaphoreType.DMA,
        pltpu.SemaphoreType.DMA,
    ],
)
def _fetch(idx_hbm, xp_hbm, my_hbm, out_x, out_y, idx_v, xbuf, outv, y_v,
           sem_x, sem_y):
    wid = lax.axis_index("s") * _NC + lax.axis_index("c")
    base = wid * _BPW
    pltpu.sync_copy(idx_hbm.at[pl.ds(base, _BPW)], idx_v)

    y_copies = []
    for j in range(_NCHUNK):
        y_copies.append(
            pltpu.async_copy(
                my_hbm.at[idx_v.at[pl.ds(j * _CHUNK, _CHUNK)]],
                y_v.at[pl.ds(j * _CHUNK, _CHUNK)],
                sem_y,
            )
        )

    x_copies = []
    for j in range(_NCHUNK):
        x_copies.append(
            pltpu.async_copy(
                xp_hbm.at[idx_v.at[pl.ds(j * _CHUNK, _CHUNK)]],
                xbuf.at[pl.ds(j * _CHUNK, _CHUNK), :],
                sem_x,
            )
        )
    for c in x_copies:
        c.wait()

    iota16 = lax.iota(jnp.int32, _LANES)

    def ebody(k, carry):
        row16 = k * _LANES + iota16
        opos = row16 * _XDIM
        for c in range(_XDIM):
            v = plsc.load_gather(xbuf, [row16, lax.broadcast(c, (_LANES,))])
            plsc.store_scatter(outv, [opos + c], v)
        return carry

    lax.fori_loop(0, _NVEC, ebody, None)

    for c in y_copies:
        c.wait()
    pltpu.sync_copy(outv, out_x.at[pl.ds(wid * _BPW * _XDIM, _BPW * _XDIM)])
    pltpu.sync_copy(y_v, out_y.at[pl.ds(base, _BPW)])


def kernel(inputs, idx, mems_x, mems_y):
    del inputs
    xp = jnp.pad(mems_x, ((0, 0), (0, _PAD - _XDIM)))
    out_xf, res_y = _fetch(idx, xp, mems_y)
    return (out_xf.reshape(_BSZ, _XDIM), res_y)
